# Initial kernel scaffold; baseline (speedup 1.0000x reference)
import jax
import jax.numpy as jnp
from jax.experimental import pallas as pl


def kernel(input_ids, counts, values, io_flags, positions, emb0, emb1, factor1, value_w, count_w, pos_w, io_w):
    raise NotImplementedError("write your pallas kernel here")


# speedup vs baseline: 8.8492x; 8.8492x over previous
import functools

import jax
import jax.numpy as jnp
from jax import lax
from jax.experimental import pallas as pl
from jax.experimental.pallas import tpu as pltpu
from jax.experimental.pallas import tpu_sc as plsc

B, L, HIDDEN = 1024, 200, 128
VOCAB = 100000
NB0 = 50000
BL = B * L


_BLK = 2000
_NBLK0 = NB0 // _BLK


def _big_table_body(emb0_ref, emb1_ref, factor1_ref, out_ref):
    i = pl.program_id(0)

    @pl.when(i < _NBLK0)
    def _copy():
        out_ref[...] = emb0_ref[...]

    @pl.when(i >= _NBLK0)
    def _proj():
        out_ref[...] = jnp.dot(emb1_ref[...], factor1_ref[...],
                               preferred_element_type=jnp.float32)


def _build_big_table(emb0, emb1, factor1):
    return pl.pallas_call(
        _big_table_body,
        grid=(2 * _NBLK0,),
        in_specs=[
            pl.BlockSpec((_BLK, HIDDEN), lambda i: (jnp.minimum(i, _NBLK0 - 1), 0)),
            pl.BlockSpec((_BLK, 32), lambda i: (jnp.maximum(i - _NBLK0, 0), 0)),
            pl.BlockSpec((32, HIDDEN), lambda i: (0, 0)),
        ],
        out_specs=pl.BlockSpec((_BLK, HIDDEN), lambda i: (i, 0)),
        out_shape=jax.ShapeDtypeStruct((VOCAB, HIDDEN), jnp.float32),
    )(emb0, emb1, factor1)


def _small_table_body(count_ref, value_ref, io_ref, out_ref):
    s = (count_ref[...][:, None, None, :]
         + value_ref[...][None, :, None, :]
         + io_ref[...][None, None, :, :])
    out_ref[...] = s.reshape(675, HIDDEN)


def _build_small_table(count_w, value_w, io_w):
    return pl.pallas_call(
        _small_table_body,
        out_shape=jax.ShapeDtypeStruct((675, HIDDEN), jnp.float32),
    )(count_w, value_w, io_w)


_NW = 32
_TPW = BL // _NW
_CH = 128
_NCH = _TPW // _CH


def _sc_body(big_hbm, s_hbm, posw_hbm, ids_hbm, cnt_hbm, val_hbm, io_hbm,
             pos_hbm, out_hbm,
             ids_v, vci_v, pos_v, cnt_v, val_v, io_v,
             brow, srow, prow, sem0, sem1, sem2):
    wid = lax.axis_index("s") * 2 + lax.axis_index("c")
    wbase = wid * _TPW

    def chunk_body(c, carry):
        base = wbase + c * _CH
        pltpu.sync_copy(ids_hbm.at[pl.ds(base, _CH)], ids_v)
        pltpu.sync_copy(cnt_hbm.at[pl.ds(base, _CH)], cnt_v)
        pltpu.sync_copy(val_hbm.at[pl.ds(base, _CH)], val_v)
        pltpu.sync_copy(io_hbm.at[pl.ds(base, _CH)], io_v)
        pltpu.sync_copy(pos_hbm.at[pl.ds(base, _CH)], pos_v)

        def vci_body(j, carry2):
            sl = pl.ds(j * 16, 16)
            vci_v[sl] = cnt_v[sl] * 45 + val_v[sl] * 3 + io_v[sl]
            return carry2

        lax.fori_loop(0, _CH // 16, vci_body, 0)

        cp0 = pltpu.async_copy(big_hbm.at[ids_v], brow, sem0)
        cp1 = pltpu.async_copy(s_hbm.at[vci_v], srow, sem1)
        cp2 = pltpu.async_copy(posw_hbm.at[pos_v], prow, sem2)
        cp0.wait()
        cp1.wait()
        cp2.wait()

        def row_body(t, carry2):
            for j in range(HIDDEN // 16):
                sl = pl.ds(j * 16, 16)
                brow[t, sl] = brow[t, sl] + srow[t, sl] + prow[t, sl]
            return carry2

        lax.fori_loop(0, _CH, row_body, 0)
        pltpu.sync_copy(brow, out_hbm.at[pl.ds(base, _CH)])
        return carry

    lax.fori_loop(0, _NCH, chunk_body, 0)


_sc_gather = functools.partial(
    pl.kernel,
    out_type=jax.ShapeDtypeStruct((BL, HIDDEN), jnp.float32),
    mesh=plsc.VectorSubcoreMesh(core_axis_name="c", subcore_axis_name="s"),
    scratch_types=[
        pltpu.VMEM((_CH,), jnp.int32),
        pltpu.VMEM((_CH,), jnp.int32),
        pltpu.VMEM((_CH,), jnp.int32),
        pltpu.VMEM((_CH,), jnp.int32),
        pltpu.VMEM((_CH,), jnp.int32),
        pltpu.VMEM((_CH,), jnp.int32),
        pltpu.VMEM((_CH, HIDDEN), jnp.float32),
        pltpu.VMEM((_CH, HIDDEN), jnp.float32),
        pltpu.VMEM((_CH, HIDDEN), jnp.float32),
        pltpu.SemaphoreType.DMA,
        pltpu.SemaphoreType.DMA,
        pltpu.SemaphoreType.DMA,
    ],
)(_sc_body)


def kernel(input_ids, counts, values, io_flags, positions,
           emb0, emb1, factor1, value_w, count_w, pos_w, io_w):
    big = _build_big_table(emb0, emb1, factor1)
    s_tab = _build_small_table(count_w, value_w, io_w)
    out = _sc_gather(big, s_tab, pos_w,
                     input_ids.reshape(BL), counts.reshape(BL),
                     values.reshape(BL), io_flags.reshape(BL),
                     positions.reshape(BL))
    return out.reshape(B, L, HIDDEN)
